# Initial kernel scaffold; baseline (speedup 1.0000x reference)
import jax
import jax.numpy as jnp
from jax.experimental import pallas as pl


def kernel(x, edge_index, batch, W_gcn, b_gcn, Wz_x, bz_x, Wz_h, bz_h, Wr_x, br_x, Wr_h, br_h, Wh_x, bh_x, Wh_h, bh_h, W_lin, b_lin):
    raise NotImplementedError("write your pallas kernel here")


# speedup vs baseline: 13.4499x; 13.4499x over previous
import jax
import jax.numpy as jnp
from jax import lax
from jax.experimental import pallas as pl
from jax.experimental.pallas import tpu as pltpu
from jax.experimental.pallas import tpu_sc as plsc

N = 10000
NP = 10240
E = 320000
EP = 323584
D = 128
OUT = 3
CH = 128
NSUB = 16
RPT = NP // NSUB
NCH_R = RPT // CH
EPS = EP // 2
EPT_A = EPS // NSUB
NCH_A = EPT_A // CH
EPT_D = EP // NSUB
NCH_D = EPT_D // CH

_f32 = jnp.float32


def _sc_mesh():
    return plsc.VectorSubcoreMesh(
        core_axis_name="c", subcore_axis_name="s", num_cores=2, num_subcores=16
    )


def _deg_call(nidx, ones128, zeros128):
    def body(nidx_ref, ones_ref, zeros_ref, out_ref, spm, ones_v, idx_v, buf_v):
        c = lax.axis_index("c")
        s = lax.axis_index("s")
        pltpu.sync_copy(zeros_ref, buf_v)
        for j in range(NCH_R):
            pltpu.sync_copy(buf_v, spm.at[pl.ds(s * RPT + j * CH, CH)])
        pltpu.sync_copy(ones_ref, ones_v)
        plsc.subcore_barrier()

        def chunk(j, carry):
            pltpu.sync_copy(
                nidx_ref.at[pl.ds(c * EP + s * EPT_D + j * CH, CH)], idx_v
            )
            pltpu.sync_copy(ones_v, spm.at[idx_v], add=True)
            return carry

        lax.fori_loop(0, NCH_D, chunk, 0)
        plsc.subcore_barrier()
        for j in range(NCH_R):
            pltpu.sync_copy(spm.at[pl.ds(s * RPT + j * CH, CH)], buf_v)
            pltpu.sync_copy(
                buf_v, out_ref.at[pl.ds(c * NP + s * RPT + j * CH, CH)]
            )

    return pl.kernel(
        body,
        out_type=jax.ShapeDtypeStruct((2 * NP, D), _f32),
        mesh=_sc_mesh(),
        scratch_types=[
            pltpu.VMEM_SHARED((NP, D), _f32),
            pltpu.VMEM((CH, D), _f32),
            pltpu.VMEM((CH,), jnp.int32),
            pltpu.VMEM((CH, D), _f32),
        ],
    )(nidx, ones128, zeros128)


def _agg_call(tbl, srcp, dstp, zeros128):
    def body(tbl_ref, src_ref, dst_ref, zeros_ref, out_ref,
             spm, isrc, idst, rows_v, sem):
        c = lax.axis_index("c")
        s = lax.axis_index("s")
        pltpu.sync_copy(zeros_ref, rows_v)
        for j in range(NCH_R):
            pltpu.sync_copy(rows_v, spm.at[pl.ds(s * RPT + j * CH, CH)])
        plsc.subcore_barrier()

        def chunk(j, carry):
            eb = c * EPS + s * EPT_A + j * CH
            pltpu.sync_copy(src_ref.at[pl.ds(eb, CH)], isrc)
            pltpu.sync_copy(dst_ref.at[pl.ds(eb, CH)], idst)
            pltpu.async_copy(tbl_ref.at[isrc], rows_v, sem).wait()
            pltpu.sync_copy(rows_v, spm.at[idst], add=True)
            return carry

        lax.fori_loop(0, NCH_A, chunk, 0)
        plsc.subcore_barrier()
        for j in range(NCH_R):
            pltpu.sync_copy(spm.at[pl.ds(s * RPT + j * CH, CH)], rows_v)
            pltpu.sync_copy(
                rows_v, out_ref.at[pl.ds(c * NP + s * RPT + j * CH, CH)]
            )

    return pl.kernel(
        body,
        out_type=jax.ShapeDtypeStruct((2 * NP, D), _f32),
        mesh=_sc_mesh(),
        scratch_types=[
            pltpu.VMEM_SHARED((NP, D), _f32),
            pltpu.VMEM((CH,), jnp.int32),
            pltpu.VMEM((CH,), jnp.int32),
            pltpu.VMEM((CH, D), _f32),
            pltpu.SemaphoreType.DMA,
        ],
    )(tbl, srcp, dstp, zeros128)


R = 512
GB = NP // R


def _dot(a, b):
    return jnp.dot(a, b, preferred_element_type=_f32,
                   precision=lax.Precision.HIGHEST)


def _tc1_call(xp, w_gcn, degd, degs):
    def body(x_ref, w_ref, dd_ref, ds_ref, g_ref, sc_ref):
        h = _dot(x_ref[...], w_ref[...])
        dinv = lax.rsqrt(dd_ref[:, 0:1] + 2.0)
        dl = ds_ref[:, 0:1]
        dinv_l = jnp.where(dl > 0.0, lax.rsqrt(jnp.maximum(dl, 1e-12)), 0.0)
        g_ref[...] = dinv * h
        cols = lax.broadcasted_iota(jnp.int32, (R, D), 1)
        sc_ref[...] = jnp.where(cols == 0, dinv,
                                jnp.where(cols == 1, dinv_l, 0.0))

    full = pl.BlockSpec((R, D), lambda i: (i, 0))
    return pl.pallas_call(
        body,
        grid=(GB,),
        in_specs=[
            full,
            pl.BlockSpec((D, D), lambda i: (0, 0)),
            full,
            full,
        ],
        out_specs=[full, full],
        out_shape=[
            jax.ShapeDtypeStruct((NP, D), _f32),
            jax.ShapeDtypeStruct((NP, D), _f32),
        ],
    )(xp, w_gcn, degd, degs)


def _tc2_call(pa0, pa1, g, sc, bg, wz0, wh0):
    def body(pa0_ref, pa1_ref, g_ref, sc_ref, bg_ref, wz0_ref, wh0_ref,
             u_ref, p_ref, q_ref):
        i = pl.program_id(0)
        dinv = sc_ref[:, 0:1]
        dinv_l = sc_ref[:, 1:2]
        agg = pa0_ref[...] + pa1_ref[...]
        x1 = jnp.maximum(dinv * agg + 2.0 * dinv * g_ref[...] + bg_ref[...],
                         0.0)
        rows = i * R + lax.broadcasted_iota(jnp.int32, (R, 1), 0)
        x1 = jnp.where(rows < N, x1, 0.0)
        u_ref[...] = dinv_l * x1
        p_ref[...] = _dot(x1, wz0_ref[...])
        q_ref[...] = _dot(x1, wh0_ref[...])

    full = pl.BlockSpec((R, D), lambda i: (i, 0))
    wspec = pl.BlockSpec((D, D), lambda i: (0, 0))
    return pl.pallas_call(
        body,
        grid=(GB,),
        in_specs=[full, full, full, full,
                  pl.BlockSpec((1, D), lambda i: (0, 0)), wspec, wspec],
        out_specs=[full, full, full],
        out_shape=[
            jax.ShapeDtypeStruct((NP, D), _f32),
            jax.ShapeDtypeStruct((NP, D), _f32),
            jax.ShapeDtypeStruct((NP, D), _f32),
        ],
    )(pa0, pa1, g, sc, bg, wz0, wh0)


def _tc3_call(t0, t1, sc, p, q, wz1, wh1, bz, bh, wl, bl):
    def body(t0_ref, t1_ref, sc_ref, p_ref, q_ref, wz1_ref, wh1_ref,
             bz_ref, bh_ref, wl_ref, bl_ref, o_ref):
        dinv_l = sc_ref[:, 1:2]
        t1 = -dinv_l * (t0_ref[...] + t1_ref[...])
        z = jax.nn.sigmoid(p_ref[...] + _dot(t1, wz1_ref[...]) + bz_ref[...])
        ht = jnp.tanh(q_ref[...] + _dot(t1, wh1_ref[...]) + bh_ref[...])
        x2 = jnp.maximum((1.0 - z) * ht, 0.0)
        o_ref[...] = _dot(x2, wl_ref[...]) + bl_ref[...]

    full = pl.BlockSpec((R, D), lambda i: (i, 0))
    wspec = pl.BlockSpec((D, D), lambda i: (0, 0))
    bspec = pl.BlockSpec((1, D), lambda i: (0, 0))
    return pl.pallas_call(
        body,
        grid=(GB,),
        in_specs=[full, full, full, full, full, wspec, wspec, bspec, bspec,
                  wspec, bspec],
        out_specs=full,
        out_shape=jax.ShapeDtypeStruct((NP, D), _f32),
    )(t0, t1, sc, p, q, wz1, wh1, bz, bh, wl, bl)


def kernel(x, edge_index, batch, W_gcn, b_gcn, Wz_x, bz_x, Wz_h, bz_h,
           Wr_x, br_x, Wr_h, br_h, Wh_x, bh_x, Wh_h, bh_h, W_lin, b_lin):
    xp = jnp.pad(x, ((0, NP - N), (0, 0)))
    src = edge_index[0]
    dst = edge_index[1]
    pad = jnp.full((EP - E,), NP - 1, dtype=jnp.int32)
    srcp = jnp.concatenate([src, pad])
    dstp = jnp.concatenate([dst, pad])
    nidx = jnp.concatenate([dstp, srcp])
    ones128 = jnp.ones((CH, D), _f32)
    zeros128 = jnp.zeros((CH, D), _f32)

    degtbl = _deg_call(nidx, ones128, zeros128)
    g, sc = _tc1_call(xp, W_gcn, degtbl[:NP], degtbl[NP:])

    part1 = _agg_call(g, srcp, dstp, zeros128)
    u, p, q = _tc2_call(part1[:NP], part1[NP:], g, sc,
                        b_gcn.reshape(1, D), Wz_x[0], Wh_x[0])

    part2 = _agg_call(u, srcp, dstp, zeros128)
    bz = (bz_x + bz_h).reshape(1, D)
    bh = (bh_x + bh_h).reshape(1, D)
    wl = jnp.pad(W_lin, ((0, 0), (0, D - OUT)))
    bl = jnp.pad(b_lin, (0, D - OUT)).reshape(1, D)
    outp = _tc3_call(part2[:NP], part2[NP:], sc, p, q, Wz_x[1], Wh_x[1],
                     bz, bh, wl, bl)
    return outp[:N, :OUT]
